# Initial kernel scaffold; baseline (speedup 1.0000x reference)
import jax
import jax.numpy as jnp
from jax.experimental import pallas as pl


def kernel(feats, segment_ids, W1, b1, W2, b2):
    raise NotImplementedError("write your pallas kernel here")


# speedup vs baseline: 6.6056x; 6.6056x over previous
import jax
import jax.numpy as jnp
from jax import lax
from jax.experimental import pallas as pl
from jax.experimental.pallas import tpu as pltpu

_N = 16384
_D = 256
_B = 8
_HID = _D // 4
_BR = 2048
_T = _N // _BR


def _body(feats_b, seg_b, W1_b, b1_b, W2_b, b2_b, out_b,
          fc, asum, amax, acnt, gate_s):
    i = pl.program_id(0)

    @pl.when(i == 0)
    def _init():
        asum[...] = jnp.zeros((_B, _D), jnp.float32)
        amax[...] = jnp.full((_B, _D), -jnp.inf, jnp.float32)
        acnt[...] = jnp.zeros((1, _B), jnp.float32)

    @pl.when(i < _T)
    def _pass1():
        x = feats_b[...]
        fc[pl.ds(i * _BR, _BR), :] = x
        segv = seg_b[...]
        onehot = (segv ==
                  lax.broadcasted_iota(jnp.int32, (_BR, _B), 1)
                  ).astype(jnp.float32)
        asum[...] += lax.dot_general(
            onehot, x, (((0,), (0,)), ((), ())),
            preferred_element_type=jnp.float32)
        acnt[...] += jnp.sum(onehot, axis=0)[None, :]
        for b in range(_B):
            mb = jnp.max(jnp.where(segv == b, x, -jnp.inf),
                         axis=0)
            amax[b] = jnp.maximum(amax[b], mb)

    @pl.when(i >= _T)
    def _pass2():
        @pl.when(i == _T)
        def _gate():
            counts = acnt[0, :]
            means = asum[...] / counts[:, None]
            mx = amax[...]

            def mlp(v):
                h = jnp.maximum(
                    jnp.dot(v, W1_b[...],
                            preferred_element_type=jnp.float32)
                    + b1_b[0, :][None, :], 0.0)
                return (jnp.dot(h, W2_b[...],
                                preferred_element_type=jnp.float32)
                        + b2_b[0, :][None, :])

            gate_s[...] = jax.nn.sigmoid(mlp(means) + mlp(mx))

        j = i - _T
        x = fc[pl.ds(j * _BR, _BR), :]
        segv = seg_b[...]
        onehot = (segv ==
                  lax.broadcasted_iota(jnp.int32, (_BR, _B), 1)
                  ).astype(jnp.float32)
        gtok = jnp.dot(onehot, gate_s[...],
                       preferred_element_type=jnp.float32)
        out_b[...] = x * gtok


def kernel(feats, segment_ids, W1, b1, W2, b2):
    seg = segment_ids.astype(jnp.int32)
    seg2 = seg.reshape(_N, 1)
    b1r = b1.reshape(1, _HID)
    b2r = b2.reshape(1, _D)

    grid = (2 * _T,)
    out = pl.pallas_call(
        _body,
        grid=grid,
        in_specs=[
            pl.BlockSpec((_BR, _D),
                         lambda i: (jnp.where(i < _T, i, _T - 1), 0)),
            pl.BlockSpec((_BR, 1),
                         lambda i: (jnp.where(i < _T, i, i - _T), 0)),
            pl.BlockSpec((_D, _HID), lambda i: (0, 0)),
            pl.BlockSpec((1, _HID), lambda i: (0, 0)),
            pl.BlockSpec((_HID, _D), lambda i: (0, 0)),
            pl.BlockSpec((1, _D), lambda i: (0, 0)),
        ],
        out_specs=pl.BlockSpec(
            (_BR, _D), lambda i: (jnp.where(i < _T, 0, i - _T), 0)),
        out_shape=jax.ShapeDtypeStruct((_N, _D), jnp.float32),
        scratch_shapes=[
            pltpu.VMEM((_N, _D), jnp.float32),
            pltpu.VMEM((_B, _D), jnp.float32),
            pltpu.VMEM((_B, _D), jnp.float32),
            pltpu.VMEM((1, _B), jnp.float32),
            pltpu.VMEM((_B, _D), jnp.float32),
        ],
        compiler_params=pltpu.CompilerParams(
            dimension_semantics=("arbitrary",)),
    )(feats, seg2, W1, b1r, W2, b2r)
    return out
